# baseline (speedup 1.0000x reference)
import jax
import jax.numpy as jnp
import numpy as np
from jax import lax
from jax.experimental import pallas as pl
from jax.experimental.pallas import tpu as pltpu
from jax.experimental.pallas import tpu_sc as plsc

D_MODEL = 768
NUM_EXPERTS = 64
TOP_K = 8
NUM_BUCKETS = 64
BUCKET_SIZE = 8
N_TOKENS = 32768
CLAMP_MIN = -10000.0
CLAMP_MAX = 10000.0

BT = 4096

NC = 2
NS = 16
NW = NC * NS
TOK_PER_W = N_TOKENS // NW


_THR = np.log([k / (64.0 - k) for k in range(1, 64)])
_TLO = np.concatenate([[-1e30], _THR]).astype(np.float32).reshape(64, 1)
_THI = np.concatenate([_THR, [1e30]]).astype(np.float32).reshape(64, 1)


def _rne_bf16(x):
    u = lax.bitcast_convert_type(x, jnp.uint32)
    u = (u + jnp.uint32(0x7FFF) + ((u >> jnp.uint32(16)) & jnp.uint32(1)))
    u = u & jnp.uint32(0xFFFF0000)
    return lax.bitcast_convert_type(u, jnp.float32)


def _tc_body(x_ref, w_ref, b_ref, wh0_ref, wh1_ref, ht0_ref, ht1_ref,
             tlo_ref, thi_ref, logits_ref, cand_ref):
    xb = x_ref[...]
    acc = lax.dot_general(
        w_ref[...], xb, (((1,), (1,)), ((), ())),
        preferred_element_type=jnp.float32,
    )
    logits_ref[...] = jnp.clip(acc + b_ref[...], CLAMP_MIN, CLAMP_MAX)

    xq = _rne_bf16(xb)
    tlo = tlo_ref[...]
    thi = thi_ref[...]
    rows = []
    for wh_ref, ht_ref in ((wh0_ref, ht0_ref), (wh1_ref, ht1_ref)):
        hl = lax.dot_general(
            _rne_bf16(wh_ref[...].reshape(1, D_MODEL)), xq,
            (((1,), (1,)), ((), ())),
            preferred_element_type=jnp.float32,
        )
        onehot = ((hl >= tlo) & (hl < thi)).astype(jnp.float32)
        ht = ht_ref[...].reshape(NUM_BUCKETS, BUCKET_SIZE).astype(jnp.float32)
        ch = lax.dot_general(
            ht, onehot, (((0,), (0,)), ((), ())),
            preferred_element_type=jnp.float32,
        )
        rows.append(ch)
    cand_ref[...] = jnp.concatenate(rows, axis=0).astype(jnp.int32)


@jax.jit
def _tc_stage(x, W, bias2d, wh0, wh1, ht0f, ht1f):
    grid = (N_TOKENS // BT,)
    return pl.pallas_call(
        _tc_body,
        grid=grid,
        in_specs=[
            pl.BlockSpec((BT, D_MODEL), lambda i: (i, 0)),
            pl.BlockSpec((NUM_EXPERTS, D_MODEL), lambda i: (0, 0)),
            pl.BlockSpec((NUM_EXPERTS, 1), lambda i: (0, 0)),
            pl.BlockSpec((1, 1, D_MODEL), lambda i: (0, 0, 0)),
            pl.BlockSpec((1, 1, D_MODEL), lambda i: (1, 0, 0)),
            pl.BlockSpec((1, NUM_BUCKETS, BUCKET_SIZE), lambda i: (0, 0, 0)),
            pl.BlockSpec((1, NUM_BUCKETS, BUCKET_SIZE), lambda i: (1, 0, 0)),
            pl.BlockSpec((NUM_BUCKETS, 1), lambda i: (0, 0)),
            pl.BlockSpec((NUM_BUCKETS, 1), lambda i: (0, 0)),
        ],
        out_specs=[
            pl.BlockSpec((NUM_EXPERTS, BT), lambda i: (0, i)),
            pl.BlockSpec((16, BT), lambda i: (0, i)),
        ],
        out_shape=[
            jax.ShapeDtypeStruct((NUM_EXPERTS, N_TOKENS), jnp.float32),
            jax.ShapeDtypeStruct((16, N_TOKENS), jnp.int32),
        ],
        compiler_params=pltpu.CompilerParams(
            dimension_semantics=("parallel",),
        ),
    )(x, W, bias2d, wh0, wh1, ht0f, ht1f,
      jnp.asarray(_TLO), jnp.asarray(_THI))


@jax.jit
def _tc_stage_raw(x, W, bias2d, Wh, hash_tables):
    wh3 = Wh.reshape(2, 1, D_MODEL)
    return _tc_stage(x, W, bias2d, wh3, wh3, hash_tables, hash_tables)


def _sc_body(logits_hbm, cand_hbm, outp_hbm, outi_hbm,
             logits_v, cand_v, outp_v, outi_v):
    wid = lax.axis_index("s") * NC + lax.axis_index("c")
    base = wid * TOK_PER_W
    pltpu.sync_copy(logits_hbm.at[:, pl.ds(base, TOK_PER_W)], logits_v)
    pltpu.sync_copy(cand_hbm.at[:, pl.ds(base, TOK_PER_W)], cand_v)

    lane = lax.iota(jnp.int32, 16)
    topmask = lane < TOP_K

    @plsc.parallel_loop(0, TOK_PER_W, step=1, unroll=8)
    def body(t):
        tv0 = jnp.full((16,), t, dtype=jnp.int32)
        cand16 = plsc.load_gather(cand_v, [lane, tv0])
        sc = plsc.load_gather(logits_v, [cand16, tv0])
        ks, vs = plsc.sort_key_val(sc, cand16, descending=True)
        m = ks[0]
        e = jnp.where(topmask, jnp.exp(ks - m), 0.0)
        s = jnp.sum(e)
        plsc.store_scatter(outp_v, [lane, tv0], e / s, mask=topmask)
        plsc.store_scatter(outi_v, [lane, tv0], vs, mask=topmask)

    plsc.subcore_barrier()
    pltpu.sync_copy(outp_v, outp_hbm.at[:, pl.ds(base, TOK_PER_W)])
    pltpu.sync_copy(outi_v, outi_hbm.at[:, pl.ds(base, TOK_PER_W)])


_sc_stage = jax.jit(pl.kernel(
    _sc_body,
    out_type=(
        jax.ShapeDtypeStruct((TOP_K, N_TOKENS), jnp.float32),
        jax.ShapeDtypeStruct((TOP_K, N_TOKENS), jnp.int32),
    ),
    mesh=plsc.VectorSubcoreMesh(core_axis_name="c", subcore_axis_name="s"),
    compiler_params=pltpu.CompilerParams(needs_layout_passes=False),
    scratch_types=[
        pltpu.VMEM((NUM_EXPERTS, TOK_PER_W), jnp.float32),
        pltpu.VMEM((16, TOK_PER_W), jnp.int32),
        pltpu.VMEM((TOP_K, TOK_PER_W), jnp.float32),
        pltpu.VMEM((TOP_K, TOK_PER_W), jnp.int32),
    ],
))


def kernel(x, W, expert_bias, Wh, hash_tables, return_raw_logits=1):
    bias2d = expert_bias.reshape(NUM_EXPERTS, 1)
    logits_t, cand_t = _tc_stage_raw(x, W, bias2d, Wh, hash_tables)
    p_t, i_t = _sc_stage(logits_t, cand_t)
    logits = logits_t.T
    return (logits, logits, logits, p_t.T, i_t.T)


# speedup vs baseline: 1.0982x; 1.0982x over previous
import jax
import jax.numpy as jnp
import numpy as np
from jax import lax
from jax.experimental import pallas as pl
from jax.experimental.pallas import tpu as pltpu
from jax.experimental.pallas import tpu_sc as plsc

D_MODEL = 768
NUM_EXPERTS = 64
TOP_K = 8
NUM_BUCKETS = 64
BUCKET_SIZE = 8
N_TOKENS = 32768
CLAMP_MIN = -10000.0
CLAMP_MAX = 10000.0

BT = 4096

NC = 2
NS = 16
NW = NC * NS
TOK_PER_W = N_TOKENS // NW


_THR = np.log([k / (64.0 - k) for k in range(1, 64)])
_TLO = np.concatenate([[-1e30], _THR]).astype(np.float32).reshape(64, 1)
_THI = np.concatenate([_THR, [1e30]]).astype(np.float32).reshape(64, 1)


def _rne_bf16(x):
    u = lax.bitcast_convert_type(x, jnp.uint32)
    u = (u + jnp.uint32(0x7FFF) + ((u >> jnp.uint32(16)) & jnp.uint32(1)))
    u = u & jnp.uint32(0xFFFF0000)
    return lax.bitcast_convert_type(u, jnp.float32)


def _tc_body(x_ref, w_ref, b_ref, wh0_ref, wh1_ref, ht0_ref, ht1_ref,
             tlo_ref, thi_ref, logits_ref, logits2_ref, logits3_ref, cand_ref):
    xb = x_ref[...]
    acc = lax.dot_general(
        w_ref[...], xb, (((1,), (1,)), ((), ())),
        preferred_element_type=jnp.float32,
    )
    logits = jnp.clip(acc + b_ref[...], CLAMP_MIN, CLAMP_MAX)
    logits_ref[...] = logits
    logits2_ref[...] = logits
    logits3_ref[...] = logits

    xq = _rne_bf16(xb)
    tlo = tlo_ref[...]
    thi = thi_ref[...]
    rows = []
    for wh_ref, ht_ref in ((wh0_ref, ht0_ref), (wh1_ref, ht1_ref)):
        hl = lax.dot_general(
            _rne_bf16(wh_ref[...].reshape(1, D_MODEL)), xq,
            (((1,), (1,)), ((), ())),
            preferred_element_type=jnp.float32,
        )
        onehot = ((hl >= tlo) & (hl < thi)).astype(jnp.float32)
        ht = ht_ref[...].reshape(NUM_BUCKETS, BUCKET_SIZE).astype(jnp.float32)
        ch = lax.dot_general(
            ht, onehot, (((0,), (0,)), ((), ())),
            preferred_element_type=jnp.float32,
        )
        rows.append(ch)
    cand_ref[...] = jnp.concatenate(rows, axis=0).astype(jnp.int32)


@jax.jit
def _tc_stage(x, W, bias2d, wh0, wh1, ht0f, ht1f):
    grid = (N_TOKENS // BT,)
    return pl.pallas_call(
        _tc_body,
        grid=grid,
        in_specs=[
            pl.BlockSpec((BT, D_MODEL), lambda i: (i, 0)),
            pl.BlockSpec((NUM_EXPERTS, D_MODEL), lambda i: (0, 0)),
            pl.BlockSpec((NUM_EXPERTS, 1), lambda i: (0, 0)),
            pl.BlockSpec((1, 1, D_MODEL), lambda i: (0, 0, 0)),
            pl.BlockSpec((1, 1, D_MODEL), lambda i: (1, 0, 0)),
            pl.BlockSpec((1, NUM_BUCKETS, BUCKET_SIZE), lambda i: (0, 0, 0)),
            pl.BlockSpec((1, NUM_BUCKETS, BUCKET_SIZE), lambda i: (1, 0, 0)),
            pl.BlockSpec((NUM_BUCKETS, 1), lambda i: (0, 0)),
            pl.BlockSpec((NUM_BUCKETS, 1), lambda i: (0, 0)),
        ],
        out_specs=[
            pl.BlockSpec((NUM_EXPERTS, BT), lambda i: (0, i)),
            pl.BlockSpec((NUM_EXPERTS, BT), lambda i: (0, i)),
            pl.BlockSpec((NUM_EXPERTS, BT), lambda i: (0, i)),
            pl.BlockSpec((16, BT), lambda i: (0, i)),
        ],
        out_shape=[
            jax.ShapeDtypeStruct((NUM_EXPERTS, N_TOKENS), jnp.float32),
            jax.ShapeDtypeStruct((NUM_EXPERTS, N_TOKENS), jnp.float32),
            jax.ShapeDtypeStruct((NUM_EXPERTS, N_TOKENS), jnp.float32),
            jax.ShapeDtypeStruct((16, N_TOKENS), jnp.int32),
        ],
        compiler_params=pltpu.CompilerParams(
            dimension_semantics=("parallel",),
        ),
    )(x, W, bias2d, wh0, wh1, ht0f, ht1f,
      jnp.asarray(_TLO), jnp.asarray(_THI))


@jax.jit
def _tc_stage_raw(x, W, bias2d, Wh, hash_tables):
    wh3 = Wh.reshape(2, 1, D_MODEL)
    return _tc_stage(x, W, bias2d, wh3, wh3, hash_tables, hash_tables)


def _sc_body(logits_hbm, cand_hbm, outp_hbm, outi_hbm,
             logits_v, cand_v, outp_v, outi_v):
    wid = lax.axis_index("s") * NC + lax.axis_index("c")
    base = wid * TOK_PER_W
    pltpu.sync_copy(logits_hbm.at[:, pl.ds(base, TOK_PER_W)], logits_v)
    pltpu.sync_copy(cand_hbm.at[:, pl.ds(base, TOK_PER_W)], cand_v)

    lane = lax.iota(jnp.int32, 16)
    topmask = lane < TOP_K

    @plsc.parallel_loop(0, TOK_PER_W, step=1, unroll=8)
    def body(t):
        tv0 = jnp.full((16,), t, dtype=jnp.int32)
        cand16 = plsc.load_gather(cand_v, [lane, tv0])
        sc = plsc.load_gather(logits_v, [cand16, tv0])
        ks, vs = plsc.sort_key_val(sc, cand16, descending=True)
        m = ks[0]
        e = jnp.where(topmask, jnp.exp(ks - m), 0.0)
        s = jnp.sum(e)
        plsc.store_scatter(outp_v, [lane, tv0], e / s, mask=topmask)
        plsc.store_scatter(outi_v, [lane, tv0], vs, mask=topmask)

    plsc.subcore_barrier()
    pltpu.sync_copy(outp_v, outp_hbm.at[:, pl.ds(base, TOK_PER_W)])
    pltpu.sync_copy(outi_v, outi_hbm.at[:, pl.ds(base, TOK_PER_W)])


_sc_stage = jax.jit(pl.kernel(
    _sc_body,
    out_type=(
        jax.ShapeDtypeStruct((TOP_K, N_TOKENS), jnp.float32),
        jax.ShapeDtypeStruct((TOP_K, N_TOKENS), jnp.int32),
    ),
    mesh=plsc.VectorSubcoreMesh(core_axis_name="c", subcore_axis_name="s"),
    compiler_params=pltpu.CompilerParams(needs_layout_passes=False),
    scratch_types=[
        pltpu.VMEM((NUM_EXPERTS, TOK_PER_W), jnp.float32),
        pltpu.VMEM((16, TOK_PER_W), jnp.int32),
        pltpu.VMEM((TOP_K, TOK_PER_W), jnp.float32),
        pltpu.VMEM((TOP_K, TOK_PER_W), jnp.int32),
    ],
))


def kernel(x, W, expert_bias, Wh, hash_tables, return_raw_logits=1):
    bias2d = expert_bias.reshape(NUM_EXPERTS, 1)
    l1_t, l2_t, l3_t, cand_t = _tc_stage_raw(x, W, bias2d, Wh, hash_tables)
    p_t, i_t = _sc_stage(l1_t, cand_t)
    return (l1_t.T, l2_t.T, l3_t.T, p_t.T, i_t.T)
